# baseline (speedup 1.0000x reference)
import functools

import jax
import jax.numpy as jnp
from jax.experimental import pallas as pl
from jax.experimental.pallas import tpu as pltpu

_H = 16


def _head_sum_matrix(F, H):
    a = jax.lax.broadcasted_iota(jnp.int32, (F, H), 0) // (F // H)
    b = jax.lax.broadcasted_iota(jnp.int32, (F, H), 1)
    return (a == b).astype(jnp.bfloat16)


def _head_widen_matrix(H, F):
    a = jax.lax.broadcasted_iota(jnp.int32, (H, F), 0)
    b = jax.lax.broadcasted_iota(jnp.int32, (H, F), 1) // (F // H)
    return (a == b).astype(jnp.float32)


def _proj_body(H, A, xs_ref, Wsub_ref, bsub_ref, Wq_ref, bq_ref,
               Wk_ref, bk_ref, Wv_ref, bv_ref, z_ref, v_ref,
               Wsub_b, Wqkv_b):
    F = Wq_ref.shape[2]
    Fv = Wv_ref.shape[2]

    @pl.when(pl.program_id(2) == 0)
    def _():
        Wsub_b[...] = Wsub_ref[0].astype(jnp.bfloat16)
        Wqkv_b[:, :F] = Wq_ref[0].astype(jnp.bfloat16)
        Wqkv_b[:, F:2 * F] = Wk_ref[0].astype(jnp.bfloat16)
        Wqkv_b[:, 2 * F:] = Wv_ref[0].astype(jnp.bfloat16)

    xt = xs_ref[...].astype(jnp.bfloat16)
    xact = jnp.maximum(
        jnp.dot(xt, Wsub_b[...], preferred_element_type=jnp.float32)
        + bsub_ref[0], 0.0).astype(jnp.bfloat16)
    qkv = jnp.dot(xact, Wqkv_b[...], preferred_element_type=jnp.float32)
    q = qkv[:, :F] + bq_ref[0]
    k = qkv[:, F:2 * F] + bk_ref[0]
    v = qkv[:, 2 * F:] + bv_ref[0]
    z = jnp.dot((q * k).astype(jnp.bfloat16), _head_sum_matrix(F, H),
                preferred_element_type=jnp.float32) * (A ** -0.5)
    z_ref[0] = z
    v_ref[0] = v.astype(jnp.bfloat16)


def _combine_body(E, H, Fv, mask_ref, z_ref, v_ref, out_ref):
    m = mask_ref[...]
    z0 = z_ref[0]
    zmax = jnp.zeros_like(z0)
    for j in range(E):
        zmax = jnp.maximum(zmax, z_ref[j] * m[:, j][:, None])
    den = 1.0
    for j in range(E):
        den = den + m[:, j][:, None] * jnp.exp(z_ref[j] - zmax)
    widen = _head_widen_matrix(H, Fv)
    acc = jnp.zeros((m.shape[0], Fv), jnp.float32)
    for j in range(E):
        mj = m[:, j][:, None]
        w = mj * mj * jnp.exp(z_ref[j] - zmax) / den
        acc = acc + v_ref[j].astype(jnp.float32) * jnp.dot(
            w, widen, preferred_element_type=jnp.float32)
    out_ref[:, :Fv] = acc
    out_ref[:, Fv:] = m


def kernel(x, Wsub, bsub, Wq, bq, Wk, bk, Wv, bv):
    E, D, _ = Wsub.shape
    H = _H
    F = Wq.shape[2]
    A = F // H
    Fv = Wv.shape[2]
    B = x.shape[0]
    mask = x[:, E * D:]
    bt = 512
    while B % bt:
        bt //= 2
    nb = B // bt

    ncores = 2 if nb % 2 == 0 else 1
    nh = nb // ncores
    z_all, v_all = pl.pallas_call(
        functools.partial(_proj_body, H, A),
        grid=(ncores, E, nh),
        in_specs=[
            pl.BlockSpec((bt, D), lambda c, e, i: (c * nh + i, e)),
            pl.BlockSpec((1, D, D), lambda c, e, i: (e, 0, 0)),
            pl.BlockSpec((1, 1, D), lambda c, e, i: (e, 0, 0)),
            pl.BlockSpec((1, D, F), lambda c, e, i: (e, 0, 0)),
            pl.BlockSpec((1, 1, F), lambda c, e, i: (e, 0, 0)),
            pl.BlockSpec((1, D, F), lambda c, e, i: (e, 0, 0)),
            pl.BlockSpec((1, 1, F), lambda c, e, i: (e, 0, 0)),
            pl.BlockSpec((1, D, Fv), lambda c, e, i: (e, 0, 0)),
            pl.BlockSpec((1, 1, Fv), lambda c, e, i: (e, 0, 0)),
        ],
        out_specs=[
            pl.BlockSpec((1, bt, H), lambda c, e, i: (e, c * nh + i, 0)),
            pl.BlockSpec((1, bt, Fv), lambda c, e, i: (e, c * nh + i, 0)),
        ],
        out_shape=[
            jax.ShapeDtypeStruct((E, B, H), jnp.float32),
            jax.ShapeDtypeStruct((E, B, Fv), jnp.bfloat16),
        ],
        scratch_shapes=[
            pltpu.VMEM((D, D), jnp.bfloat16),
            pltpu.VMEM((D, 2 * F + Fv), jnp.bfloat16),
        ],
        compiler_params=pltpu.CompilerParams(
            dimension_semantics=("parallel", "arbitrary", "arbitrary")),
    )(x, Wsub, bsub.reshape(E, 1, D), Wq, bq.reshape(E, 1, F),
      Wk, bk.reshape(E, 1, F), Wv, bv.reshape(E, 1, Fv))

    bt2 = min(bt, 512)
    nb2 = B // bt2
    out = pl.pallas_call(
        functools.partial(_combine_body, E, H, Fv),
        grid=(nb2,),
        in_specs=[
            pl.BlockSpec((bt2, E), lambda i: (i, 0)),
            pl.BlockSpec((E, bt2, H), lambda i: (0, i, 0)),
            pl.BlockSpec((E, bt2, Fv), lambda i: (0, i, 0)),
        ],
        out_specs=pl.BlockSpec((bt2, Fv + E), lambda i: (i, 0)),
        out_shape=jax.ShapeDtypeStruct((B, Fv + E), jnp.float32),
        compiler_params=pltpu.CompilerParams(
            dimension_semantics=("arbitrary",)),
    )(mask, z_all, v_all)
    return out


# speedup vs baseline: 1.0547x; 1.0547x over previous
import functools

import jax
import jax.numpy as jnp
from jax.experimental import pallas as pl
from jax.experimental.pallas import tpu as pltpu

_H = 16


def _head_sum_matrix(F, H):
    a = jax.lax.broadcasted_iota(jnp.int32, (F, H), 0) // (F // H)
    b = jax.lax.broadcasted_iota(jnp.int32, (F, H), 1)
    return (a == b).astype(jnp.bfloat16)


def _head_widen_matrix(H, F):
    a = jax.lax.broadcasted_iota(jnp.int32, (H, F), 0)
    b = jax.lax.broadcasted_iota(jnp.int32, (H, F), 1) // (F // H)
    return (a == b).astype(jnp.float32)


def _proj_body(H, A, xs_ref, Wsub_ref, bsub_ref, Wq_ref, bq_ref,
               Wk_ref, bk_ref, Wv_ref, bv_ref, z_ref, v_ref,
               Wsub_b, Wqkv_b):
    F = Wq_ref.shape[2]
    Fv = Wv_ref.shape[2]

    @pl.when(pl.program_id(1) == 0)
    def _():
        Wsub_b[...] = Wsub_ref[0].astype(jnp.bfloat16)
        Wqkv_b[:, :F] = Wq_ref[0].astype(jnp.bfloat16)
        Wqkv_b[:, F:2 * F] = Wk_ref[0].astype(jnp.bfloat16)
        Wqkv_b[:, 2 * F:] = Wv_ref[0].astype(jnp.bfloat16)

    bt = xs_ref.shape[0]
    hc = bt // 2 if bt % 2 == 0 else bt
    for h in range(bt // hc):
        sl = pl.ds(h * hc, hc)
        xt = xs_ref[sl, :].astype(jnp.bfloat16)
        xact = jnp.maximum(
            jnp.dot(xt, Wsub_b[...], preferred_element_type=jnp.float32)
            + bsub_ref[0], 0.0).astype(jnp.bfloat16)
        qkv = jnp.dot(xact, Wqkv_b[...], preferred_element_type=jnp.float32)
        q = qkv[:, :F] + bq_ref[0]
        k = qkv[:, F:2 * F] + bk_ref[0]
        v = qkv[:, 2 * F:] + bv_ref[0]
        z = jnp.dot((q * k).astype(jnp.bfloat16), _head_sum_matrix(F, H),
                    preferred_element_type=jnp.float32) * (A ** -0.5)
        z_ref[0, sl, :] = z
        v_ref[0, sl, :] = v.astype(jnp.bfloat16)


def _combine_body(E, H, Fv, mask_ref, z_ref, v_ref, out_ref):
    m = mask_ref[...]
    z0 = z_ref[0]
    zmax = jnp.zeros_like(z0)
    for j in range(E):
        zmax = jnp.maximum(zmax, z_ref[j] * m[:, j][:, None])
    den = 1.0
    for j in range(E):
        den = den + m[:, j][:, None] * jnp.exp(z_ref[j] - zmax)
    widen = _head_widen_matrix(H, Fv)
    acc = jnp.zeros((m.shape[0], Fv), jnp.float32)
    for j in range(E):
        mj = m[:, j][:, None]
        w = mj * mj * jnp.exp(z_ref[j] - zmax) / den
        acc = acc + v_ref[j].astype(jnp.float32) * jnp.dot(
            w, widen, preferred_element_type=jnp.float32)
    out_ref[:, :Fv] = acc
    out_ref[:, Fv:] = m


def kernel(x, Wsub, bsub, Wq, bq, Wk, bk, Wv, bv):
    E, D, _ = Wsub.shape
    H = _H
    F = Wq.shape[2]
    A = F // H
    Fv = Wv.shape[2]
    B = x.shape[0]
    mask = x[:, E * D:]
    bt = 512
    while B % bt:
        bt //= 2
    nb = B // bt

    z_all, v_all = pl.pallas_call(
        functools.partial(_proj_body, H, A),
        grid=(E, nb),
        in_specs=[
            pl.BlockSpec((bt, D), lambda e, i: (i, e)),
            pl.BlockSpec((1, D, D), lambda e, i: (e, 0, 0)),
            pl.BlockSpec((1, 1, D), lambda e, i: (e, 0, 0)),
            pl.BlockSpec((1, D, F), lambda e, i: (e, 0, 0)),
            pl.BlockSpec((1, 1, F), lambda e, i: (e, 0, 0)),
            pl.BlockSpec((1, D, F), lambda e, i: (e, 0, 0)),
            pl.BlockSpec((1, 1, F), lambda e, i: (e, 0, 0)),
            pl.BlockSpec((1, D, Fv), lambda e, i: (e, 0, 0)),
            pl.BlockSpec((1, 1, Fv), lambda e, i: (e, 0, 0)),
        ],
        out_specs=[
            pl.BlockSpec((1, bt, H), lambda e, i: (e, i, 0)),
            pl.BlockSpec((1, bt, Fv), lambda e, i: (e, i, 0)),
        ],
        out_shape=[
            jax.ShapeDtypeStruct((E, B, H), jnp.float32),
            jax.ShapeDtypeStruct((E, B, Fv), jnp.bfloat16),
        ],
        scratch_shapes=[
            pltpu.VMEM((D, D), jnp.bfloat16),
            pltpu.VMEM((D, 2 * F + Fv), jnp.bfloat16),
        ],
        compiler_params=pltpu.CompilerParams(
            dimension_semantics=("arbitrary", "arbitrary")),
    )(x, Wsub, bsub.reshape(E, 1, D), Wq, bq.reshape(E, 1, F),
      Wk, bk.reshape(E, 1, F), Wv, bv.reshape(E, 1, Fv))

    bt2 = min(bt, 512)
    nb2 = B // bt2
    out = pl.pallas_call(
        functools.partial(_combine_body, E, H, Fv),
        grid=(nb2,),
        in_specs=[
            pl.BlockSpec((bt2, E), lambda i: (i, 0)),
            pl.BlockSpec((E, bt2, H), lambda i: (0, i, 0)),
            pl.BlockSpec((E, bt2, Fv), lambda i: (0, i, 0)),
        ],
        out_specs=pl.BlockSpec((bt2, Fv + E), lambda i: (i, 0)),
        out_shape=jax.ShapeDtypeStruct((B, Fv + E), jnp.float32),
        compiler_params=pltpu.CompilerParams(
            dimension_semantics=("arbitrary",)),
    )(mask, z_all, v_all)
    return out
